# baseline (speedup 1.0000x reference)
import functools

import numpy as np
import jax
import jax.numpy as jnp
from jax import lax
from jax.experimental import pallas as pl
from jax.experimental.pallas import tpu as pltpu
from jax.experimental.pallas import tpu_sc as plsc

N = 100000
E = 3200000
NPAD = 102400
NWORDS = NPAD // 16
NPASS = 25
PSZ = NPAD // NPASS
PSLICE = PSZ // 16
PVEC = PSLICE // 16
NWORKERS = 32
EPW = E // NWORKERS
CHUNK = 2000
NCHUNK = EPW // CHUNK
VPC = CHUNK // 16

BINS = 256
R0 = 0.1
INV_DR = BINS / 2.0
QSTRIDE = 264
TABW = 16 * QSTRIDE

_C = np.array([0.02817, 0.28022, 0.50986, 0.18175], dtype=np.float64)
_D = np.array([0.20162, 0.4029, 0.94229, 3.1998], dtype=np.float64)


def _build_energy_table() -> np.ndarray:
    z = np.array([1.0, 6.0, 7.0, 8.0], dtype=np.float64)
    rcov = np.array([0.31, 0.76, 0.71, 0.66], dtype=np.float64)
    p, a0 = 0.23, 0.4685
    r = R0 + np.arange(BINS + 1) / INV_DR
    tab = np.zeros((16, QSTRIDE), dtype=np.float64)
    for ti in range(4):
        for tj in range(4):
            q = ti * 4 + tj
            zi, zj = z[ti], z[tj]
            rc = rcov[ti] + rcov[tj]
            a = a0 / (zi ** p + zj ** p)
            da = _D / a
            factor = 14.399645478425668 * zi * zj

            def phi(x):
                return np.sum(_C * np.exp(-np.multiply.outer(x, da)), axis=-1)

            def dphi(x):
                return np.sum(-_C * da * np.exp(-np.multiply.outer(x, da)), axis=-1)

            def d2phi(x):
                return np.sum(_C * da * da * np.exp(-np.multiply.outer(x, da)), axis=-1)

            e = factor / r * phi(r)
            ec = factor / rc * phi(rc)
            dec = factor / rc * (-phi(rc) / rc + dphi(rc))
            d2ec = factor / rc * (d2phi(rc) - 2.0 / rc * dphi(rc)
                                  + 2.0 * phi(rc) / rc ** 2)
            A = (-3.0 * dec + rc * d2ec) / rc ** 2
            B = (2.0 * dec - rc * d2ec) / rc ** 3
            Cc = -ec + rc * dec / 2.0 - rc * rc * d2ec / 12.0
            e = 0.5 * (e + A / 3.0 * r ** 3 + B / 4.0 * r ** 4 + Cc)
            tab[q, :BINS + 1] = np.where(r > rc, 0.0, e)
    import ml_dtypes
    f32 = tab.astype(np.float32)
    e0b = f32.astype(ml_dtypes.bfloat16).view(np.uint16).astype(np.uint32)
    e1b = (np.roll(f32, -1, axis=1).astype(ml_dtypes.bfloat16)
           .view(np.uint16).astype(np.uint32))
    return (e0b | (e1b << 16)).view(np.int32).reshape(-1)


_ETAB = _build_energy_table()


@functools.cache
def _make_zbl_sc():
    mesh = plsc.VectorSubcoreMesh(core_axis_name="c", subcore_axis_name="s",
                                  num_cores=2, num_subcores=16)
    return pl.kernel(
        _zbl_sc,
        out_type=jax.ShapeDtypeStruct((2 * NPAD,), jnp.float32),
        mesh=mesh,
        scratch_types=[
            pltpu.VMEM((NPAD,), jnp.float32),
            pltpu.VMEM((NWORDS,), jnp.int32),
            pltpu.VMEM((TABW,), jnp.int32),
            pltpu.VMEM((CHUNK,), jnp.int32),
            pltpu.VMEM((CHUNK,), jnp.int32),
            pltpu.VMEM((CHUNK,), jnp.float32),
            pltpu.VMEM((CHUNK,), jnp.int32),
            pltpu.VMEM((CHUNK,), jnp.int32),
            pltpu.VMEM((CHUNK,), jnp.float32),
            pltpu.VMEM_SHARED((16 * PSZ,), jnp.float32),
            pltpu.SemaphoreType.DMA,
            pltpu.SemaphoreType.DMA,
        ],
        compiler_params=pltpu.CompilerParams(needs_layout_passes=False),
    )


def _zbl_sc(rij_hbm, edge_hbm, tpack_hbm, tab_hbm, out_hbm,
            acc, tpack, tab, srcA, dstA, rijA, srcB, dstB, rijB,
            shared, semA, semB):
    cid = lax.axis_index("c")
    sid = lax.axis_index("s")
    wid = cid * 16 + sid

    pltpu.sync_copy(tpack_hbm, tpack)
    pltpu.sync_copy(tab_hbm, tab)

    def _zero(i, _):
        for u in range(8):
            acc[pl.ds(i * 128 + u * 16, 16)] = jnp.zeros((16,), jnp.float32)
        return _

    lax.fori_loop(0, NPAD // 128, _zero, None)

    base = wid * EPW
    last_off = base + (NCHUNK - 1) * CHUNK

    def _start(off, sb, db, rb, sem):
        pltpu.async_copy(edge_hbm.at[pl.ds(off, CHUNK)], sb, sem)
        pltpu.async_copy(edge_hbm.at[pl.ds(E + off, CHUNK)], db, sem)
        pltpu.async_copy(rij_hbm.at[pl.ds(off, CHUNK)], rb, sem)

    def _wait(off, sb, db, rb, sem):
        pltpu.make_async_copy(edge_hbm.at[pl.ds(off, CHUNK)], sb, sem).wait()
        pltpu.make_async_copy(edge_hbm.at[pl.ds(E + off, CHUNK)], db, sem).wait()
        pltpu.make_async_copy(rij_hbm.at[pl.ds(off, CHUNK)], rb, sem).wait()

    UNROLL = 5

    def _compute(sb, db, rb):
        def _vec(v, __):
            o = v * (16 * UNROLL)
            for k in range(UNROLL):
                s = sb[pl.ds(o + k * 16, 16)]
                t = db[pl.ds(o + k * 16, 16)]
                r = rb[pl.ds(o + k * 16, 16)]
                wi = plsc.load_gather(tpack, [s >> 4])
                wj = plsc.load_gather(tpack, [t >> 4])
                ti = (wi >> ((s & 15) << 1)) & 3
                tj = (wj >> ((t & 15) << 1)) & 3
                q = (ti << 2) | tj
                u = (r - np.float32(R0)) * np.float32(INV_DR)
                b = jnp.minimum(u.astype(jnp.int32), BINS - 1)
                frac = u - b.astype(jnp.float32)
                idx = q * QSTRIDE + b
                g = plsc.load_gather(tab, [idx])
                e0 = plsc.bitcast(g << 16, jnp.float32)
                e1 = plsc.bitcast(g & jnp.int32(-65536), jnp.float32)
                e = e0 + frac * (e1 - e0)
                plsc.addupdate_scatter(acc, [s], e)
            return __

        lax.fori_loop(0, VPC // UNROLL, _vec, None)

    _start(base, srcA, dstA, rijA, semA)

    def _pair(i, _):
        offA = base + (2 * i) * CHUNK
        offB = offA + CHUNK
        offA2 = lax.min(offA + 2 * CHUNK, last_off)
        _start(offB, srcB, dstB, rijB, semB)
        _wait(offA, srcA, dstA, rijA, semA)
        _compute(srcA, dstA, rijA)
        _start(offA2, srcA, dstA, rijA, semA)
        _wait(offB, srcB, dstB, rijB, semB)
        _compute(srcB, dstB, rijB)
        return _

    lax.fori_loop(0, NCHUNK // 2, _pair, None)
    _wait(last_off, srcA, dstA, rijA, semA)

    for p in range(NPASS):
        pbase = p * PSZ
        pltpu.sync_copy(acc.at[pl.ds(pbase, PSZ)],
                        shared.at[pl.ds(sid * PSZ, PSZ)])
        plsc.subcore_barrier()
        for t in range(16):
            pltpu.sync_copy(shared.at[pl.ds(t * PSZ + sid * PSLICE, PSLICE)],
                            acc.at[pl.ds(pbase + t * PSLICE, PSLICE)])

        def _red(v, _):
            o = pbase + v * 16
            tot = acc[pl.ds(o, 16)]
            for t in range(1, 16):
                tot = tot + acc[pl.ds(t * PSLICE + o, 16)]
            acc[pl.ds(o, 16)] = tot
            return _

        lax.fori_loop(0, PVEC, _red, None)
        pltpu.sync_copy(
            acc.at[pl.ds(pbase, PSLICE)],
            out_hbm.at[pl.ds(cid * NPAD + pbase + sid * PSLICE, PSLICE)])
        plsc.subcore_barrier()


def _tc_sum_body(p_ref, o_ref):
    o_ref[...] = p_ref[0] + p_ref[1]


_tc_sum = pl.pallas_call(
    _tc_sum_body,
    out_shape=jax.ShapeDtypeStruct((NPAD // 128, 128), jnp.float32),
)


def kernel(rij, types, edge_index):
    types = types.astype(jnp.int32)
    edge_index = edge_index.astype(jnp.int32)
    rij = rij.astype(jnp.float32)
    tpad = jnp.zeros((NPAD,), jnp.int32).at[:N].set(types).reshape(NWORDS, 16)
    shifts = (jnp.arange(16, dtype=jnp.int32) * 2)[None, :]
    tpack = jnp.sum(tpad << shifts, axis=1, dtype=jnp.int32)
    tab = jnp.asarray(_ETAB)
    partials = _make_zbl_sc()(rij, edge_index.reshape(2 * E), tpack, tab)
    out = _tc_sum(partials.reshape(2, NPAD // 128, 128))
    return out.reshape(NPAD)[:N]


# speedup vs baseline: 1.7171x; 1.7171x over previous
import functools

import numpy as np
import jax
import jax.numpy as jnp
from jax import lax
from jax.experimental import pallas as pl
from jax.experimental.pallas import tpu as pltpu
from jax.experimental.pallas import tpu_sc as plsc

N = 100000
E = 3200000
NPAD = 102400
NWORDS = NPAD // 16
NPASS = 25
PSZ = NPAD // NPASS
PSLICE = PSZ // 16
PVEC = PSLICE // 16
NWORKERS = 32
EPW = E // NWORKERS
CHUNK = 2000
NCHUNK = EPW // CHUNK
VPC = CHUNK // 16

BINS = 256
R0 = 0.1
INV_DR = BINS / 2.0
QSTRIDE = 264
TABW = 16 * QSTRIDE

_C = np.array([0.02817, 0.28022, 0.50986, 0.18175], dtype=np.float64)
_D = np.array([0.20162, 0.4029, 0.94229, 3.1998], dtype=np.float64)


def _build_energy_table() -> np.ndarray:
    z = np.array([1.0, 6.0, 7.0, 8.0], dtype=np.float64)
    rcov = np.array([0.31, 0.76, 0.71, 0.66], dtype=np.float64)
    p, a0 = 0.23, 0.4685
    r = R0 + np.arange(BINS + 1) / INV_DR
    tab = np.zeros((16, QSTRIDE), dtype=np.float64)
    for ti in range(4):
        for tj in range(4):
            q = ti * 4 + tj
            zi, zj = z[ti], z[tj]
            rc = rcov[ti] + rcov[tj]
            a = a0 / (zi ** p + zj ** p)
            da = _D / a
            factor = 14.399645478425668 * zi * zj

            def phi(x):
                return np.sum(_C * np.exp(-np.multiply.outer(x, da)), axis=-1)

            def dphi(x):
                return np.sum(-_C * da * np.exp(-np.multiply.outer(x, da)), axis=-1)

            def d2phi(x):
                return np.sum(_C * da * da * np.exp(-np.multiply.outer(x, da)), axis=-1)

            e = factor / r * phi(r)
            ec = factor / rc * phi(rc)
            dec = factor / rc * (-phi(rc) / rc + dphi(rc))
            d2ec = factor / rc * (d2phi(rc) - 2.0 / rc * dphi(rc)
                                  + 2.0 * phi(rc) / rc ** 2)
            A = (-3.0 * dec + rc * d2ec) / rc ** 2
            B = (2.0 * dec - rc * d2ec) / rc ** 3
            Cc = -ec + rc * dec / 2.0 - rc * rc * d2ec / 12.0
            e = 0.5 * (e + A / 3.0 * r ** 3 + B / 4.0 * r ** 4 + Cc)
            tab[q, :BINS + 1] = np.where(r > rc, 0.0, e)
    import ml_dtypes
    f32 = tab.astype(np.float32)
    e0b = f32.astype(ml_dtypes.bfloat16).view(np.uint16).astype(np.uint32)
    e1b = (np.roll(f32, -1, axis=1).astype(ml_dtypes.bfloat16)
           .view(np.uint16).astype(np.uint32))
    return (e0b | (e1b << 16)).view(np.int32).reshape(-1)


_ETAB = _build_energy_table()


@functools.cache
def _make_zbl_sc():
    mesh = plsc.VectorSubcoreMesh(core_axis_name="c", subcore_axis_name="s",
                                  num_cores=2, num_subcores=16)
    return pl.kernel(
        _zbl_sc,
        out_type=jax.ShapeDtypeStruct((2 * NPAD,), jnp.float32),
        mesh=mesh,
        scratch_types=[
            pltpu.VMEM((NPAD,), jnp.float32),
            pltpu.VMEM((NWORDS,), jnp.int32),
            pltpu.VMEM((TABW,), jnp.int32),
            pltpu.VMEM((CHUNK,), jnp.int32),
            pltpu.VMEM((CHUNK,), jnp.int32),
            pltpu.VMEM((CHUNK,), jnp.float32),
            pltpu.VMEM((CHUNK,), jnp.int32),
            pltpu.VMEM((CHUNK,), jnp.int32),
            pltpu.VMEM((CHUNK,), jnp.float32),
            pltpu.VMEM_SHARED((16 * PSZ,), jnp.float32),
            pltpu.SemaphoreType.DMA,
            pltpu.SemaphoreType.DMA,
        ],
        compiler_params=pltpu.CompilerParams(needs_layout_passes=False),
    )


def _zbl_sc(rij_hbm, edge_hbm, tpack_hbm, tab_hbm, out_hbm,
            acc, tpack, tab, srcA, dstA, rijA, srcB, dstB, rijB,
            shared, semA, semB):
    cid = lax.axis_index("c")
    sid = lax.axis_index("s")
    wid = cid * 16 + sid

    pltpu.sync_copy(tpack_hbm, tpack)
    pltpu.sync_copy(tab_hbm, tab)

    def _zero(i, _):
        for u in range(8):
            acc[pl.ds(i * 128 + u * 16, 16)] = jnp.zeros((16,), jnp.float32)
        return _

    lax.fori_loop(0, NPAD // 128, _zero, None)

    base = wid * EPW
    last_off = base + (NCHUNK - 1) * CHUNK

    def _start(off, sb, db, rb, sem):
        pltpu.async_copy(edge_hbm.at[pl.ds(off, CHUNK)], sb, sem)
        pltpu.async_copy(edge_hbm.at[pl.ds(E + off, CHUNK)], db, sem)
        pltpu.async_copy(rij_hbm.at[pl.ds(off, CHUNK)], rb, sem)

    def _wait(off, sb, db, rb, sem):
        pltpu.make_async_copy(edge_hbm.at[pl.ds(off, CHUNK)], sb, sem).wait()
        pltpu.make_async_copy(edge_hbm.at[pl.ds(E + off, CHUNK)], db, sem).wait()
        pltpu.make_async_copy(rij_hbm.at[pl.ds(off, CHUNK)], rb, sem).wait()

    def _compute(sb, db, rb):
        @plsc.parallel_loop(0, VPC, unroll=5)
        def _vec(v):
            o = v * 16
            s = sb[pl.ds(o, 16)]
            t = db[pl.ds(o, 16)]
            r = rb[pl.ds(o, 16)]
            wi = plsc.load_gather(tpack, [s >> 4])
            wj = plsc.load_gather(tpack, [t >> 4])
            ti = (wi >> ((s & 15) << 1)) & 3
            tj = (wj >> ((t & 15) << 1)) & 3
            q = (ti << 2) | tj
            u = (r - np.float32(R0)) * np.float32(INV_DR)
            b = jnp.minimum(u.astype(jnp.int32), BINS - 1)
            frac = u - b.astype(jnp.float32)
            idx = q * QSTRIDE + b
            g = plsc.load_gather(tab, [idx])
            e0 = plsc.bitcast(g << 16, jnp.float32)
            e1 = plsc.bitcast(g & jnp.int32(-65536), jnp.float32)
            e = e0 + frac * (e1 - e0)
            plsc.addupdate_scatter(acc, [s], e)

    _start(base, srcA, dstA, rijA, semA)

    def _pair(i, _):
        offA = base + (2 * i) * CHUNK
        offB = offA + CHUNK
        offA2 = lax.min(offA + 2 * CHUNK, last_off)
        _start(offB, srcB, dstB, rijB, semB)
        _wait(offA, srcA, dstA, rijA, semA)
        _compute(srcA, dstA, rijA)
        _start(offA2, srcA, dstA, rijA, semA)
        _wait(offB, srcB, dstB, rijB, semB)
        _compute(srcB, dstB, rijB)
        return _

    lax.fori_loop(0, NCHUNK // 2, _pair, None)
    _wait(last_off, srcA, dstA, rijA, semA)

    for p in range(NPASS):
        pbase = p * PSZ
        pltpu.sync_copy(acc.at[pl.ds(pbase, PSZ)],
                        shared.at[pl.ds(sid * PSZ, PSZ)])
        plsc.subcore_barrier()
        for t in range(16):
            pltpu.sync_copy(shared.at[pl.ds(t * PSZ + sid * PSLICE, PSLICE)],
                            acc.at[pl.ds(pbase + t * PSLICE, PSLICE)])

        def _red(v, _):
            o = pbase + v * 16
            tot = acc[pl.ds(o, 16)]
            for t in range(1, 16):
                tot = tot + acc[pl.ds(t * PSLICE + o, 16)]
            acc[pl.ds(o, 16)] = tot
            return _

        lax.fori_loop(0, PVEC, _red, None)
        pltpu.sync_copy(
            acc.at[pl.ds(pbase, PSLICE)],
            out_hbm.at[pl.ds(cid * NPAD + pbase + sid * PSLICE, PSLICE)])
        plsc.subcore_barrier()


def _tc_sum_body(p_ref, o_ref):
    o_ref[...] = p_ref[0] + p_ref[1]


_tc_sum = pl.pallas_call(
    _tc_sum_body,
    out_shape=jax.ShapeDtypeStruct((NPAD // 128, 128), jnp.float32),
)


def kernel(rij, types, edge_index):
    types = types.astype(jnp.int32)
    edge_index = edge_index.astype(jnp.int32)
    rij = rij.astype(jnp.float32)
    tpad = jnp.zeros((NPAD,), jnp.int32).at[:N].set(types).reshape(NWORDS, 16)
    shifts = (jnp.arange(16, dtype=jnp.int32) * 2)[None, :]
    tpack = jnp.sum(tpad << shifts, axis=1, dtype=jnp.int32)
    tab = jnp.asarray(_ETAB)
    partials = _make_zbl_sc()(rij, edge_index.reshape(2 * E), tpack, tab)
    out = _tc_sum(partials.reshape(2, NPAD // 128, 128))
    return out.reshape(NPAD)[:N]


# speedup vs baseline: 1.7483x; 1.0182x over previous
import functools

import numpy as np
import jax
import jax.numpy as jnp
from jax import lax
from jax.experimental import pallas as pl
from jax.experimental.pallas import tpu as pltpu
from jax.experimental.pallas import tpu_sc as plsc

N = 100000
E = 3200000
NPAD = 102400
NWORDS = NPAD // 16
NPASS = 25
PSZ = NPAD // NPASS
PSLICE = PSZ // 16
PVEC = PSLICE // 16
NWORKERS = 32
EPW = E // NWORKERS
CHUNK = 2000
NCHUNK = EPW // CHUNK
VPC = CHUNK // 16

BINS = 256
R0 = 0.1
INV_DR = BINS / 2.0
QSTRIDE = 264
TABW = 16 * QSTRIDE

_C = np.array([0.02817, 0.28022, 0.50986, 0.18175], dtype=np.float64)
_D = np.array([0.20162, 0.4029, 0.94229, 3.1998], dtype=np.float64)


def _build_energy_table() -> np.ndarray:
    z = np.array([1.0, 6.0, 7.0, 8.0], dtype=np.float64)
    rcov = np.array([0.31, 0.76, 0.71, 0.66], dtype=np.float64)
    p, a0 = 0.23, 0.4685
    r = R0 + np.arange(BINS + 1) / INV_DR
    tab = np.zeros((16, QSTRIDE), dtype=np.float64)
    for ti in range(4):
        for tj in range(4):
            q = ti * 4 + tj
            zi, zj = z[ti], z[tj]
            rc = rcov[ti] + rcov[tj]
            a = a0 / (zi ** p + zj ** p)
            da = _D / a
            factor = 14.399645478425668 * zi * zj

            def phi(x):
                return np.sum(_C * np.exp(-np.multiply.outer(x, da)), axis=-1)

            def dphi(x):
                return np.sum(-_C * da * np.exp(-np.multiply.outer(x, da)), axis=-1)

            def d2phi(x):
                return np.sum(_C * da * da * np.exp(-np.multiply.outer(x, da)), axis=-1)

            e = factor / r * phi(r)
            ec = factor / rc * phi(rc)
            dec = factor / rc * (-phi(rc) / rc + dphi(rc))
            d2ec = factor / rc * (d2phi(rc) - 2.0 / rc * dphi(rc)
                                  + 2.0 * phi(rc) / rc ** 2)
            A = (-3.0 * dec + rc * d2ec) / rc ** 2
            B = (2.0 * dec - rc * d2ec) / rc ** 3
            Cc = -ec + rc * dec / 2.0 - rc * rc * d2ec / 12.0
            e = 0.5 * (e + A / 3.0 * r ** 3 + B / 4.0 * r ** 4 + Cc)
            tab[q, :BINS + 1] = np.where(r > rc, 0.0, e)
    import ml_dtypes
    f32 = tab.astype(np.float32)
    e0b = f32.astype(ml_dtypes.bfloat16).view(np.uint16).astype(np.uint32)
    e1b = (np.roll(f32, -1, axis=1).astype(ml_dtypes.bfloat16)
           .view(np.uint16).astype(np.uint32))
    return (e0b | (e1b << 16)).view(np.int32).reshape(-1)


_ETAB = _build_energy_table()


@functools.cache
def _make_zbl_sc():
    mesh = plsc.VectorSubcoreMesh(core_axis_name="c", subcore_axis_name="s",
                                  num_cores=2, num_subcores=16)
    return pl.kernel(
        _zbl_sc,
        out_type=jax.ShapeDtypeStruct((2 * NPAD,), jnp.float32),
        mesh=mesh,
        scratch_types=[
            pltpu.VMEM((NPAD,), jnp.float32),
            pltpu.VMEM((NWORDS,), jnp.int32),
            pltpu.VMEM((TABW,), jnp.int32),
            pltpu.VMEM((CHUNK,), jnp.int32),
            pltpu.VMEM((CHUNK,), jnp.int32),
            pltpu.VMEM((CHUNK,), jnp.float32),
            pltpu.VMEM((CHUNK,), jnp.int32),
            pltpu.VMEM((CHUNK,), jnp.int32),
            pltpu.VMEM((CHUNK,), jnp.float32),
            pltpu.VMEM_SHARED((16 * PSZ,), jnp.float32),
            pltpu.SemaphoreType.DMA,
            pltpu.SemaphoreType.DMA,
        ],
        compiler_params=pltpu.CompilerParams(needs_layout_passes=False),
    )


def _zbl_sc(rij_hbm, edge_hbm, tpack_hbm, tab_hbm, out_hbm,
            acc, tpack, tab, srcA, dstA, rijA, srcB, dstB, rijB,
            shared, semA, semB):
    cid = lax.axis_index("c")
    sid = lax.axis_index("s")
    wid = cid * 16 + sid

    pltpu.sync_copy(tpack_hbm, tpack)
    pltpu.sync_copy(tab_hbm, tab)

    @plsc.parallel_loop(0, NPAD // 16, unroll=8)
    def _zero(i):
        acc[pl.ds(i * 16, 16)] = jnp.zeros((16,), jnp.float32)

    base = wid * EPW
    last_off = base + (NCHUNK - 1) * CHUNK

    def _start(off, sb, db, rb, sem):
        pltpu.async_copy(edge_hbm.at[pl.ds(off, CHUNK)], sb, sem)
        pltpu.async_copy(edge_hbm.at[pl.ds(E + off, CHUNK)], db, sem)
        pltpu.async_copy(rij_hbm.at[pl.ds(off, CHUNK)], rb, sem)

    def _wait(off, sb, db, rb, sem):
        pltpu.make_async_copy(edge_hbm.at[pl.ds(off, CHUNK)], sb, sem).wait()
        pltpu.make_async_copy(edge_hbm.at[pl.ds(E + off, CHUNK)], db, sem).wait()
        pltpu.make_async_copy(rij_hbm.at[pl.ds(off, CHUNK)], rb, sem).wait()

    def _compute(sb, db, rb):
        @plsc.parallel_loop(0, VPC, unroll=5)
        def _vec(v):
            o = v * 16
            s = sb[pl.ds(o, 16)]
            t = db[pl.ds(o, 16)]
            r = rb[pl.ds(o, 16)]
            wi = plsc.load_gather(tpack, [s >> 4])
            wj = plsc.load_gather(tpack, [t >> 4])
            ti = (wi >> ((s & 15) << 1)) & 3
            tj = (wj >> ((t & 15) << 1)) & 3
            q = (ti << 2) | tj
            u = (r - np.float32(R0)) * np.float32(INV_DR)
            b = jnp.minimum(u.astype(jnp.int32), BINS - 1)
            frac = u - b.astype(jnp.float32)
            idx = q * QSTRIDE + b
            g = plsc.load_gather(tab, [idx])
            e0 = plsc.bitcast(g << 16, jnp.float32)
            e1 = plsc.bitcast(g & jnp.int32(-65536), jnp.float32)
            e = e0 + frac * (e1 - e0)
            plsc.addupdate_scatter(acc, [s], e)

    _start(base, srcA, dstA, rijA, semA)

    def _pair(i, _):
        offA = base + (2 * i) * CHUNK
        offB = offA + CHUNK
        offA2 = lax.min(offA + 2 * CHUNK, last_off)
        _start(offB, srcB, dstB, rijB, semB)
        _wait(offA, srcA, dstA, rijA, semA)
        _compute(srcA, dstA, rijA)
        _start(offA2, srcA, dstA, rijA, semA)
        _wait(offB, srcB, dstB, rijB, semB)
        _compute(srcB, dstB, rijB)
        return _

    lax.fori_loop(0, NCHUNK // 2, _pair, None)
    _wait(last_off, srcA, dstA, rijA, semA)

    for p in range(NPASS):
        pbase = p * PSZ
        pltpu.sync_copy(acc.at[pl.ds(pbase, PSZ)],
                        shared.at[pl.ds(sid * PSZ, PSZ)])
        plsc.subcore_barrier()
        for t in range(16):
            pltpu.sync_copy(shared.at[pl.ds(t * PSZ + sid * PSLICE, PSLICE)],
                            acc.at[pl.ds(pbase + t * PSLICE, PSLICE)])

        @plsc.parallel_loop(0, PVEC, unroll=4)
        def _red(v):
            o = pbase + v * 16
            tot = acc[pl.ds(o, 16)]
            for t in range(1, 16):
                tot = tot + acc[pl.ds(t * PSLICE + o, 16)]
            acc[pl.ds(o, 16)] = tot
        pltpu.sync_copy(
            acc.at[pl.ds(pbase, PSLICE)],
            out_hbm.at[pl.ds(cid * NPAD + pbase + sid * PSLICE, PSLICE)])
        plsc.subcore_barrier()


def _tc_sum_body(p_ref, o_ref):
    o_ref[...] = p_ref[0] + p_ref[1]


_tc_sum = pl.pallas_call(
    _tc_sum_body,
    out_shape=jax.ShapeDtypeStruct((NPAD // 128, 128), jnp.float32),
)


def kernel(rij, types, edge_index):
    types = types.astype(jnp.int32)
    edge_index = edge_index.astype(jnp.int32)
    rij = rij.astype(jnp.float32)
    tpad = jnp.zeros((NPAD,), jnp.int32).at[:N].set(types).reshape(NWORDS, 16)
    shifts = (jnp.arange(16, dtype=jnp.int32) * 2)[None, :]
    tpack = jnp.sum(tpad << shifts, axis=1, dtype=jnp.int32)
    tab = jnp.asarray(_ETAB)
    partials = _make_zbl_sc()(rij, edge_index.reshape(2 * E), tpack, tab)
    out = _tc_sum(partials.reshape(2, NPAD // 128, 128))
    return out.reshape(NPAD)[:N]


# speedup vs baseline: 2.6796x; 1.5327x over previous
import functools

import numpy as np
import jax
import jax.numpy as jnp
from jax import lax
from jax.experimental import pallas as pl
from jax.experimental.pallas import tpu as pltpu
from jax.experimental.pallas import tpu_sc as plsc

N = 100000
E = 3200000
NPAD = 102400
NWORDS = NPAD // 16
NPASS = 25
PSZ = NPAD // NPASS
PSLICE = PSZ // 16
PVEC = PSLICE // 16
NWORKERS = 32
EPW = E // NWORKERS
CHUNK = 2000
NCHUNK = EPW // CHUNK
VPC = CHUNK // 16

BINS = 256
R0 = 0.1
INV_DR = BINS / 2.0
QSTRIDE = 264
TABW = 16 * QSTRIDE

_C = np.array([0.02817, 0.28022, 0.50986, 0.18175], dtype=np.float64)
_D = np.array([0.20162, 0.4029, 0.94229, 3.1998], dtype=np.float64)


def _build_energy_table() -> np.ndarray:
    z = np.array([1.0, 6.0, 7.0, 8.0], dtype=np.float64)
    rcov = np.array([0.31, 0.76, 0.71, 0.66], dtype=np.float64)
    p, a0 = 0.23, 0.4685
    r = R0 + np.arange(BINS + 1) / INV_DR
    tab = np.zeros((16, QSTRIDE), dtype=np.float64)
    for ti in range(4):
        for tj in range(4):
            q = ti * 4 + tj
            zi, zj = z[ti], z[tj]
            rc = rcov[ti] + rcov[tj]
            a = a0 / (zi ** p + zj ** p)
            da = _D / a
            factor = 14.399645478425668 * zi * zj

            def phi(x):
                return np.sum(_C * np.exp(-np.multiply.outer(x, da)), axis=-1)

            def dphi(x):
                return np.sum(-_C * da * np.exp(-np.multiply.outer(x, da)), axis=-1)

            def d2phi(x):
                return np.sum(_C * da * da * np.exp(-np.multiply.outer(x, da)), axis=-1)

            e = factor / r * phi(r)
            ec = factor / rc * phi(rc)
            dec = factor / rc * (-phi(rc) / rc + dphi(rc))
            d2ec = factor / rc * (d2phi(rc) - 2.0 / rc * dphi(rc)
                                  + 2.0 * phi(rc) / rc ** 2)
            A = (-3.0 * dec + rc * d2ec) / rc ** 2
            B = (2.0 * dec - rc * d2ec) / rc ** 3
            Cc = -ec + rc * dec / 2.0 - rc * rc * d2ec / 12.0
            e = 0.5 * (e + A / 3.0 * r ** 3 + B / 4.0 * r ** 4 + Cc)
            tab[q, :BINS + 1] = np.where(r > rc, 0.0, e)
    import ml_dtypes
    f32 = tab.astype(np.float32)
    e0b = f32.astype(ml_dtypes.bfloat16).view(np.uint16).astype(np.uint32)
    e1b = (np.roll(f32, -1, axis=1).astype(ml_dtypes.bfloat16)
           .view(np.uint16).astype(np.uint32))
    return (e0b | (e1b << 16)).view(np.int32).reshape(-1)


_ETAB = _build_energy_table()


@functools.cache
def _make_zbl_sc():
    mesh = plsc.VectorSubcoreMesh(core_axis_name="c", subcore_axis_name="s",
                                  num_cores=2, num_subcores=16)
    return pl.kernel(
        _zbl_sc,
        out_type=jax.ShapeDtypeStruct((32 * NPAD,), jnp.float32),
        mesh=mesh,
        scratch_types=[
            pltpu.VMEM((NPAD,), jnp.float32),
            pltpu.VMEM((NWORDS,), jnp.int32),
            pltpu.VMEM((TABW,), jnp.int32),
            pltpu.VMEM((CHUNK,), jnp.int32),
            pltpu.VMEM((CHUNK,), jnp.int32),
            pltpu.VMEM((CHUNK,), jnp.float32),
            pltpu.VMEM((CHUNK,), jnp.int32),
            pltpu.VMEM((CHUNK,), jnp.int32),
            pltpu.VMEM((CHUNK,), jnp.float32),
            pltpu.SemaphoreType.DMA,
            pltpu.SemaphoreType.DMA,
        ],
        compiler_params=pltpu.CompilerParams(needs_layout_passes=False),
    )


def _zbl_sc(rij_hbm, edge_hbm, tpack_hbm, tab_hbm, out_hbm,
            acc, tpack, tab, srcA, dstA, rijA, srcB, dstB, rijB,
            semA, semB):
    cid = lax.axis_index("c")
    sid = lax.axis_index("s")
    wid = cid * 16 + sid

    pltpu.sync_copy(tpack_hbm, tpack)
    pltpu.sync_copy(tab_hbm, tab)

    @plsc.parallel_loop(0, NPAD // 16, unroll=8)
    def _zero(i):
        acc[pl.ds(i * 16, 16)] = jnp.zeros((16,), jnp.float32)

    base = wid * EPW
    last_off = base + (NCHUNK - 1) * CHUNK

    def _start(off, sb, db, rb, sem):
        pltpu.async_copy(edge_hbm.at[pl.ds(off, CHUNK)], sb, sem)
        pltpu.async_copy(edge_hbm.at[pl.ds(E + off, CHUNK)], db, sem)
        pltpu.async_copy(rij_hbm.at[pl.ds(off, CHUNK)], rb, sem)

    def _wait(off, sb, db, rb, sem):
        pltpu.make_async_copy(edge_hbm.at[pl.ds(off, CHUNK)], sb, sem).wait()
        pltpu.make_async_copy(edge_hbm.at[pl.ds(E + off, CHUNK)], db, sem).wait()
        pltpu.make_async_copy(rij_hbm.at[pl.ds(off, CHUNK)], rb, sem).wait()

    def _compute(sb, db, rb):
        @plsc.parallel_loop(0, VPC, unroll=5)
        def _vec(v):
            o = v * 16
            s = sb[pl.ds(o, 16)]
            t = db[pl.ds(o, 16)]
            r = rb[pl.ds(o, 16)]
            wi = plsc.load_gather(tpack, [s >> 4])
            wj = plsc.load_gather(tpack, [t >> 4])
            ti = (wi >> ((s & 15) << 1)) & 3
            tj = (wj >> ((t & 15) << 1)) & 3
            q = (ti << 2) | tj
            u = (r - np.float32(R0)) * np.float32(INV_DR)
            b = jnp.minimum(u.astype(jnp.int32), BINS - 1)
            frac = u - b.astype(jnp.float32)
            idx = q * QSTRIDE + b
            g = plsc.load_gather(tab, [idx])
            e0 = plsc.bitcast(g << 16, jnp.float32)
            e1 = plsc.bitcast(g & jnp.int32(-65536), jnp.float32)
            e = e0 + frac * (e1 - e0)
            plsc.addupdate_scatter(acc, [s], e)

    _start(base, srcA, dstA, rijA, semA)

    def _pair(i, _):
        offA = base + (2 * i) * CHUNK
        offB = offA + CHUNK
        offA2 = lax.min(offA + 2 * CHUNK, last_off)
        _start(offB, srcB, dstB, rijB, semB)
        _wait(offA, srcA, dstA, rijA, semA)
        _compute(srcA, dstA, rijA)
        _start(offA2, srcA, dstA, rijA, semA)
        _wait(offB, srcB, dstB, rijB, semB)
        _compute(srcB, dstB, rijB)
        return _

    lax.fori_loop(0, NCHUNK // 2, _pair, None)
    _wait(last_off, srcA, dstA, rijA, semA)

    pltpu.sync_copy(acc, out_hbm.at[pl.ds(wid * NPAD, NPAD)])


def _tc_sum_body(p_ref, o_ref):
    o_ref[...] = jnp.sum(p_ref[...], axis=0)


_tc_sum = pl.pallas_call(
    _tc_sum_body,
    out_shape=jax.ShapeDtypeStruct((NPAD // 128, 128), jnp.float32),
)


def kernel(rij, types, edge_index):
    types = types.astype(jnp.int32)
    edge_index = edge_index.astype(jnp.int32)
    rij = rij.astype(jnp.float32)
    tpad = jnp.zeros((NPAD,), jnp.int32).at[:N].set(types).reshape(NWORDS, 16)
    shifts = (jnp.arange(16, dtype=jnp.int32) * 2)[None, :]
    tpack = jnp.sum(tpad << shifts, axis=1, dtype=jnp.int32)
    tab = jnp.asarray(_ETAB)
    partials = _make_zbl_sc()(rij, edge_index.reshape(2 * E), tpack, tab)
    out = _tc_sum(partials.reshape(32, NPAD // 128, 128))
    return out.reshape(NPAD)[:N]
